# baseline (speedup 1.0000x reference)
import numpy as np
import jax
import jax.numpy as jnp
from jax.experimental import pallas as pl
from jax.experimental.pallas import tpu as pltpu

H = 96
EMBED = 16
NUM_STEPS = 4
OUT_DIM = 10
NCELL = 81
NE = 1620
DEG = 20
BB = 8


def _build_edges_np():
    idx = np.arange(81).reshape(9, 9)
    e = []
    for i in range(9):
        v = idx[i, :]
        e += [(a, b) for a in v for b in v if a != b]
        v = idx[:, i]
        e += [(a, b) for a in v for b in v if a != b]
    for i in range(3):
        for j in range(3):
            v = idx[3 * i:3 * (i + 1), 3 * j:3 * (j + 1)].reshape(-1)
            e += [(a, b) for a in v for b in v if a != b]
    e = sorted(set((int(a), int(b)) for a, b in e))
    return np.array(e, dtype=np.int64)


_EDGES = _build_edges_np()
_ORD = np.lexsort((_EDGES[:, 0], _EDGES[:, 1]))
_SRC = _EDGES[_ORD, 0]
_DST = _EDGES[_ORD, 1]
assert np.array_equal(_DST, np.repeat(np.arange(NCELL), DEG))

CHUNK = 88
NEP = DEG * CHUNK
_GR = np.zeros((NEP, 2 * NCELL), np.float32)
for _k in range(DEG):
    _d = np.arange(NCELL)
    _rows = _k * CHUNK + _d
    _GR[_rows, _SRC[_d * DEG + _k]] = 1.0
    _GR[_rows, NCELL + _d] = 1.0
_RC = np.array([(i, j) for i in range(9) for j in range(9)])
_RCOH = np.concatenate([np.eye(EMBED, dtype=np.float32)[_RC[:, 0]],
                        np.eye(EMBED, dtype=np.float32)[_RC[:, 1]]], axis=1)


def _relu(v):
    return jnp.maximum(v, 0.0)


def _dot(a, b):
    return jnp.dot(a, b, preferred_element_type=jnp.float32)


def _body(emb_ref, gr_ref,
          e2x_W1, e2x_b1, e2x_W2, e2x_b2, e2x_W3, e2x_b3,
          msg_W1a, msg_W1b, msg_b1, msg_W2, msg_b2, msg_W3, msg_b3,
          li_W1a, li_W1b, li_b1, li_W2, li_b2, li_W3, li_b3,
          lstm_Wih, lstm_Whh, lstm_bih, lstm_bhh, r2o_W, r2o_b,
          out_ref):
    emb = emb_ref[...]
    GR = gr_ref[...]

    x = _relu(_dot(emb, e2x_W1[...]) + e2x_b1[...])
    x = _relu(_dot(x, e2x_W2[...]) + e2x_b2[...])
    x = _dot(x, e2x_W3[...]) + e2x_b3[...]

    xl = _dot(x, li_W1b[...]) + li_b1[...]

    h = jnp.zeros_like(x)
    c = jnp.zeros_like(x)
    hm = x
    bsum = lstm_bih[...] + lstm_bhh[...]
    for t in range(NUM_STEPS):
        A = _dot(hm, msg_W1a[...])
        B = _dot(hm, msg_W1b[...])
        m1s = []
        for p in range(BB):
            ab = jnp.concatenate([A[p * NCELL:(p + 1) * NCELL],
                                  B[p * NCELL:(p + 1) * NCELL]], axis=0)
            m1s.append(_relu(_dot(GR, ab) + msg_b1[...]))
        m1 = jnp.concatenate(m1s, axis=0)
        m2 = _relu(_dot(m1, msg_W2[...]) + msg_b2[...])
        ts = []
        for p in range(BB):
            base = p * NEP
            acc = m2[base:base + CHUNK]
            for k in range(1, DEG):
                acc = acc + m2[base + k * CHUNK:base + k * CHUNK + CHUNK]
            ts.append(acc[:NCELL])
        T = jnp.concatenate(ts, axis=0)
        fm = _dot(T, msg_W3[...]) + DEG * msg_b3[...]

        l1 = _relu(_dot(fm, li_W1a[...]) + xl)
        l2 = _relu(_dot(l1, li_W2[...]) + li_b2[...])
        itl = _dot(l2, li_W3[...]) + li_b3[...]

        gates = _dot(itl, lstm_Wih[...]) + _dot(h, lstm_Whh[...]) + bsum
        i_g = gates[:, 0 * H:1 * H]
        f_g = gates[:, 1 * H:2 * H]
        g_g = gates[:, 2 * H:3 * H]
        o_g = gates[:, 3 * H:4 * H]
        c = jax.nn.sigmoid(f_g) * c + jax.nn.sigmoid(i_g) * jnp.tanh(g_g)
        h = jax.nn.sigmoid(o_g) * jnp.tanh(c)
        hm = h
        out_ref[t] = _dot(h, r2o_W[...]) + r2o_b[...]


def kernel(inp, e2x_W1, e2x_b1, e2x_W2, e2x_b2, e2x_W3, e2x_b3,
           msg_W1, msg_b1, msg_W2, msg_b2, msg_W3, msg_b3,
           li_W1, li_b1, li_W2, li_b2, li_W3, li_b3,
           lstm_Wih, lstm_Whh, lstm_bih, lstm_bhh, r2o_W, r2o_b):
    bs = inp.shape[0]
    assert bs % BB == 0
    n_blocks = bs // BB

    flat = inp.reshape(-1).astype(jnp.int32)
    emb = jax.nn.one_hot(flat, EMBED, dtype=jnp.float32)
    rcoh = jnp.tile(jnp.asarray(_RCOH), (bs, 1))
    embedded = jnp.concatenate([emb, rcoh], axis=1)

    def b2d(v):
        return v.reshape(1, -1)

    weights = [
        e2x_W1, b2d(e2x_b1), e2x_W2, b2d(e2x_b2), e2x_W3, b2d(e2x_b3),
        msg_W1[:H], msg_W1[H:], b2d(msg_b1), msg_W2, b2d(msg_b2), msg_W3, b2d(msg_b3),
        li_W1[:H], li_W1[H:], b2d(li_b1), li_W2, b2d(li_b2), li_W3, b2d(li_b3),
        lstm_Wih, lstm_Whh, b2d(lstm_bih), b2d(lstm_bhh), r2o_W, b2d(r2o_b),
    ]

    def fixed(shape):
        return pl.BlockSpec(shape, lambda g: (0,) * len(shape))

    in_specs = [
        pl.BlockSpec((BB * NCELL, 3 * EMBED), lambda g: (g, 0)),
        fixed((NEP, 2 * NCELL)),
    ] + [fixed(tuple(w.shape)) for w in weights]

    out = pl.pallas_call(
        _body,
        grid=(n_blocks,),
        in_specs=in_specs,
        out_specs=pl.BlockSpec((NUM_STEPS, BB * NCELL, OUT_DIM),
                               lambda g: (0, g, 0)),
        out_shape=jax.ShapeDtypeStruct((NUM_STEPS, bs * NCELL, OUT_DIM),
                                       jnp.float32),
        compiler_params=pltpu.CompilerParams(
            dimension_semantics=("parallel",)),
    )(embedded, jnp.asarray(_GR), *weights)
    return out


# speedup vs baseline: 1.1017x; 1.1017x over previous
import numpy as np
import jax
import jax.numpy as jnp
from jax.experimental import pallas as pl
from jax.experimental.pallas import tpu as pltpu

H = 96
EMBED = 16
NUM_STEPS = 4
OUT_DIM = 10
NCELL = 81
NE = 1620
DEG = 20
BB = 8


def _build_edges_np():
    idx = np.arange(81).reshape(9, 9)
    e = []
    for i in range(9):
        v = idx[i, :]
        e += [(a, b) for a in v for b in v if a != b]
        v = idx[:, i]
        e += [(a, b) for a in v for b in v if a != b]
    for i in range(3):
        for j in range(3):
            v = idx[3 * i:3 * (i + 1), 3 * j:3 * (j + 1)].reshape(-1)
            e += [(a, b) for a in v for b in v if a != b]
    e = sorted(set((int(a), int(b)) for a, b in e))
    return np.array(e, dtype=np.int64)


_EDGES = _build_edges_np()
_ORD = np.lexsort((_EDGES[:, 0], _EDGES[:, 1]))
_SRC = _EDGES[_ORD, 0]
_DST = _EDGES[_ORD, 1]
assert np.array_equal(_DST, np.repeat(np.arange(NCELL), DEG))

_GR = np.zeros((NE, 2 * NCELL), np.float32)
_GR[np.arange(NE), _SRC] = 1.0
_GR[np.arange(NE), NCELL + _DST] = 1.0
_S = np.kron(np.eye(NCELL, dtype=np.float32), np.ones((1, DEG), np.float32))
_RC = np.array([(i, j) for i in range(9) for j in range(9)])
_RCOH = np.concatenate([np.eye(EMBED, dtype=np.float32)[_RC[:, 0]],
                        np.eye(EMBED, dtype=np.float32)[_RC[:, 1]]], axis=1)


def _relu(v):
    return jnp.maximum(v, 0.0)


def _dot(a, b):
    return jnp.dot(a, b, preferred_element_type=jnp.float32)


def _body(emb_ref, gr_ref, s_ref,
          e2x_W1, e2x_b1, e2x_W2, e2x_b2, e2x_W3, e2x_b3,
          msg_W1a, msg_W1b, msg_b1, msg_W2, msg_b2, msg_W3, msg_b3,
          li_W1a, li_W1b, li_b1, li_W2, li_b2, li_W3, li_b3,
          lstm_Wih, lstm_Whh, lstm_bih, lstm_bhh, r2o_W, r2o_b,
          out_ref):
    emb = emb_ref[...]
    GR = gr_ref[...]
    S = s_ref[...]

    x = _relu(_dot(emb, e2x_W1[...]) + e2x_b1[...])
    x = _relu(_dot(x, e2x_W2[...]) + e2x_b2[...])
    x = _dot(x, e2x_W3[...]) + e2x_b3[...]

    xl = _dot(x, li_W1b[...]) + li_b1[...]

    h = jnp.zeros_like(x)
    c = jnp.zeros_like(x)
    hm = x
    bsum = lstm_bih[...] + lstm_bhh[...]
    bf = jnp.bfloat16
    W2b = msg_W2[...].astype(bf)
    for t in range(NUM_STEPS):
        A = _dot(hm, msg_W1a[...]).astype(bf)
        B = _dot(hm, msg_W1b[...]).astype(bf)
        m1s = []
        for p in range(BB):
            ab = jnp.concatenate([A[p * NCELL:(p + 1) * NCELL],
                                  B[p * NCELL:(p + 1) * NCELL]], axis=0)
            m1s.append(_relu(_dot(GR, ab) + msg_b1[...]).astype(bf))
        m1 = jnp.concatenate(m1s, axis=0)
        m2 = _relu(_dot(m1, W2b) + msg_b2[...]).astype(bf)
        ts = [_dot(S, m2[p * NE:(p + 1) * NE]) for p in range(BB)]
        T = jnp.concatenate(ts, axis=0)
        fm = _dot(T, msg_W3[...]) + DEG * msg_b3[...]

        l1 = _relu(_dot(fm, li_W1a[...]) + xl)
        l2 = _relu(_dot(l1, li_W2[...]) + li_b2[...])
        itl = _dot(l2, li_W3[...]) + li_b3[...]

        gates = _dot(itl, lstm_Wih[...]) + _dot(h, lstm_Whh[...]) + bsum
        i_g = gates[:, 0 * H:1 * H]
        f_g = gates[:, 1 * H:2 * H]
        g_g = gates[:, 2 * H:3 * H]
        o_g = gates[:, 3 * H:4 * H]
        c = jax.nn.sigmoid(f_g) * c + jax.nn.sigmoid(i_g) * jnp.tanh(g_g)
        h = jax.nn.sigmoid(o_g) * jnp.tanh(c)
        hm = h
        out_ref[t] = _dot(h, r2o_W[...]) + r2o_b[...]


def kernel(inp, e2x_W1, e2x_b1, e2x_W2, e2x_b2, e2x_W3, e2x_b3,
           msg_W1, msg_b1, msg_W2, msg_b2, msg_W3, msg_b3,
           li_W1, li_b1, li_W2, li_b2, li_W3, li_b3,
           lstm_Wih, lstm_Whh, lstm_bih, lstm_bhh, r2o_W, r2o_b):
    bs = inp.shape[0]
    assert bs % BB == 0
    n_blocks = bs // BB

    flat = inp.reshape(-1).astype(jnp.int32)
    emb = jax.nn.one_hot(flat, EMBED, dtype=jnp.float32)
    rcoh = jnp.tile(jnp.asarray(_RCOH), (bs, 1))
    embedded = jnp.concatenate([emb, rcoh], axis=1)

    def b2d(v):
        return v.reshape(1, -1)

    weights = [
        e2x_W1, b2d(e2x_b1), e2x_W2, b2d(e2x_b2), e2x_W3, b2d(e2x_b3),
        msg_W1[:H], msg_W1[H:], b2d(msg_b1), msg_W2, b2d(msg_b2), msg_W3, b2d(msg_b3),
        li_W1[:H], li_W1[H:], b2d(li_b1), li_W2, b2d(li_b2), li_W3, b2d(li_b3),
        lstm_Wih, lstm_Whh, b2d(lstm_bih), b2d(lstm_bhh), r2o_W, b2d(r2o_b),
    ]

    def fixed(shape):
        return pl.BlockSpec(shape, lambda g: (0,) * len(shape))

    in_specs = [
        pl.BlockSpec((BB * NCELL, 3 * EMBED), lambda g: (g, 0)),
        fixed((NE, 2 * NCELL)),
        fixed((NCELL, NE)),
    ] + [fixed(tuple(w.shape)) for w in weights]

    out = pl.pallas_call(
        _body,
        grid=(n_blocks,),
        in_specs=in_specs,
        out_specs=pl.BlockSpec((NUM_STEPS, BB * NCELL, OUT_DIM),
                               lambda g: (0, g, 0)),
        out_shape=jax.ShapeDtypeStruct((NUM_STEPS, bs * NCELL, OUT_DIM),
                                       jnp.float32),
        compiler_params=pltpu.CompilerParams(
            dimension_semantics=("parallel",)),
    )(embedded, jnp.asarray(_GR, jnp.bfloat16), jnp.asarray(_S, jnp.bfloat16),
      *weights)
    return out


# speedup vs baseline: 1.5938x; 1.4466x over previous
import numpy as np
import jax
import jax.numpy as jnp
from jax.experimental import pallas as pl
from jax.experimental.pallas import tpu as pltpu

H = 96
H2 = 2 * H
EMBED = 16
NUM_STEPS = 4
OUT_DIM = 10
NCELL = 81
NE = 1620
DEG = 20
PB = 8
ROWS = PB * NCELL


def _build_edges_np():
    idx = np.arange(81).reshape(9, 9)
    e = []
    for i in range(9):
        v = idx[i, :]
        e += [(a, b) for a in v for b in v if a != b]
        v = idx[:, i]
        e += [(a, b) for a in v for b in v if a != b]
    for i in range(3):
        for j in range(3):
            v = idx[3 * i:3 * (i + 1), 3 * j:3 * (j + 1)].reshape(-1)
            e += [(a, b) for a in v for b in v if a != b]
    e = sorted(set((int(a), int(b)) for a, b in e))
    return np.array(e, dtype=np.int64)


_EDGES = _build_edges_np()
_ORD = np.lexsort((_EDGES[:, 0], _EDGES[:, 1]))
_SRC = _EDGES[_ORD, 0]
_DST = _EDGES[_ORD, 1]
assert np.array_equal(_DST, np.repeat(np.arange(NCELL), DEG))

_GRA = np.zeros((NE, 2 * NCELL + 1), np.float32)
_GRA[np.arange(NE), _SRC] = 1.0
_GRA[np.arange(NE), NCELL + _DST] = 1.0
_GRA[:, 2 * NCELL] = 1.0
_S = np.kron(np.eye(NCELL, dtype=np.float32), np.ones((1, DEG), np.float32))
_RC = np.array([(i, j) for i in range(9) for j in range(9)])
_RCOH = np.concatenate([np.eye(EMBED, dtype=np.float32)[_RC[:, 0]],
                        np.eye(EMBED, dtype=np.float32)[_RC[:, 1]]], axis=1)


def _relu(v):
    return jnp.maximum(v, 0.0)


def _sig(v):
    return 0.5 * jnp.tanh(0.5 * v) + 0.5


def _dot(a, b):
    return jnp.dot(a, b, preferred_element_type=jnp.float32)


def _body(emb_ref, gra_ref, s_ref,
          e2x_W1d, e2x_b1d, e2x_W2d, e2x_b2d, e2x_W3d, e2x_b3d,
          msg_W1ad, msg_W1bd, msg_b1d,
          msg_W2d, msg_b2d, msg_W3Ld, msg_c0d,
          li_W1bd, li_W2d, li_b2d, li_W3d, li_b3d,
          lstm_WihP, lstm_WhhP, lstm_bP, r2o_Wd, r2o_bd,
          out_ref):
    emb = emb_ref[...]
    GRA = gra_ref[...]
    S = s_ref[...]

    x = _relu(_dot(emb, e2x_W1d[...]) + e2x_b1d[...])
    x = _relu(_dot(x, e2x_W2d[...]) + e2x_b2d[...])
    x = _dot(x, e2x_W3d[...]) + e2x_b3d[...]

    xc = _dot(x, li_W1bd[...]) + msg_c0d[...]

    h = jnp.zeros_like(x)
    c = jnp.zeros_like(x)
    hm = x
    b1row = msg_b1d[...]
    W2d = msg_W2d[...]
    b2d = msg_b2d[...]
    W3Ld = msg_W3Ld[...]

    for t in range(NUM_STEPS):
        A = _dot(hm, msg_W1ad[...])
        B = _dot(hm, msg_W1bd[...])
        tps = []
        for p in range(PB):
            r = p * NCELL
            ab = jnp.concatenate([A[r:r + NCELL], B[r:r + NCELL], b1row],
                                 axis=0)
            m1 = _relu(_dot(GRA, ab))
            m2 = _relu(_dot(m1, W2d) + b2d)
            tps.append(_dot(S, m2))
        T = jnp.concatenate(tps, axis=0)

        l1 = _relu(_dot(T, W3Ld) + xc)
        l2 = _relu(_dot(l1, li_W2d[...]) + li_b2d[...])
        itl = _dot(l2, li_W3d[...]) + li_b3d[...]

        gates = _dot(itl, lstm_WihP[...]) + _dot(h, lstm_WhhP[...]) + lstm_bP[...]
        i_g = gates[:, 0 * H2:1 * H2]
        f_g = gates[:, 1 * H2:2 * H2]
        g_g = gates[:, 2 * H2:3 * H2]
        o_g = gates[:, 3 * H2:4 * H2]
        c = _sig(f_g) * c + _sig(i_g) * jnp.tanh(g_g)
        h = _sig(o_g) * jnp.tanh(c)
        hm = h
        out_ref[t] = _dot(h, r2o_Wd[...]) + r2o_bd[...]


def kernel(inp, e2x_W1, e2x_b1, e2x_W2, e2x_b2, e2x_W3, e2x_b3,
           msg_W1, msg_b1, msg_W2, msg_b2, msg_W3, msg_b3,
           li_W1, li_b1, li_W2, li_b2, li_W3, li_b3,
           lstm_Wih, lstm_Whh, lstm_bih, lstm_bhh, r2o_W, r2o_b):
    bs = inp.shape[0]
    assert bs % (2 * PB) == 0
    n_blocks = bs // (2 * PB)

    flat = inp.reshape(-1).astype(jnp.int32)
    emb = jax.nn.one_hot(flat, EMBED, dtype=jnp.float32)
    rcoh = jnp.tile(jnp.asarray(_RCOH), (bs, 1))
    embedded = jnp.concatenate([emb, rcoh], axis=1)
    packed = embedded.reshape(bs // 2, 2, NCELL, 3 * EMBED)
    packed = packed.transpose(0, 2, 1, 3).reshape(bs // 2 * NCELL, 6 * EMBED)

    def b2d(v):
        return jnp.tile(v.reshape(1, -1), (1, 2))

    def bdiag(w):
        z = jnp.zeros_like(w)
        return jnp.concatenate([jnp.concatenate([w, z], axis=1),
                                jnp.concatenate([z, w], axis=1)], axis=0)

    li_W1a, li_W1b = li_W1[:H], li_W1[H:]
    W3L = msg_W3 @ li_W1a
    c0 = DEG * (msg_b3 @ li_W1a) + li_b1
    WihP = jnp.concatenate([bdiag(lstm_Wih[:, g * H:(g + 1) * H])
                            for g in range(4)], axis=1)
    WhhP = jnp.concatenate([bdiag(lstm_Whh[:, g * H:(g + 1) * H])
                            for g in range(4)], axis=1)
    bP = jnp.concatenate([b2d(lstm_bih[g * H:(g + 1) * H] +
                              lstm_bhh[g * H:(g + 1) * H])
                          for g in range(4)], axis=1)

    weights = [
        bdiag(e2x_W1), b2d(e2x_b1), bdiag(e2x_W2), b2d(e2x_b2),
        bdiag(e2x_W3), b2d(e2x_b3),
        bdiag(msg_W1[:H]), bdiag(msg_W1[H:]), b2d(msg_b1),
        bdiag(msg_W2), b2d(msg_b2), bdiag(W3L), b2d(c0),
        bdiag(li_W1b), bdiag(li_W2), b2d(li_b2), bdiag(li_W3), b2d(li_b3),
        WihP, WhhP, bP, bdiag(r2o_W), b2d(r2o_b),
    ]

    def fixed(shape):
        return pl.BlockSpec(shape, lambda g: (0,) * len(shape))

    in_specs = [
        pl.BlockSpec((ROWS, 6 * EMBED), lambda g: (g, 0)),
        fixed((NE, 2 * NCELL + 1)),
        fixed((NCELL, NE)),
    ] + [fixed(tuple(w.shape)) for w in weights]

    out = pl.pallas_call(
        _body,
        grid=(n_blocks,),
        in_specs=in_specs,
        out_specs=pl.BlockSpec((NUM_STEPS, ROWS, 2 * OUT_DIM),
                               lambda g: (0, g, 0)),
        out_shape=jax.ShapeDtypeStruct((NUM_STEPS, bs // 2 * NCELL, 2 * OUT_DIM),
                                       jnp.float32),
        compiler_params=pltpu.CompilerParams(
            dimension_semantics=("parallel",)),
    )(packed, jnp.asarray(_GRA), jnp.asarray(_S), *weights)

    o = out.reshape(NUM_STEPS, bs // 2, NCELL, 2, OUT_DIM)
    o = o.transpose(0, 1, 3, 2, 4).reshape(NUM_STEPS, bs * NCELL, OUT_DIM)
    return o
